# baseline (speedup 1.0000x reference)
import functools

import jax
import jax.numpy as jnp
from jax import lax
from jax.experimental import pallas as pl
from jax.experimental.pallas import tpu as pltpu
from jax.experimental.pallas import tpu_sc as plsc

_N = 10000
_E = 160000
_R = 3
_NB = 8
_TILES = 16
_EPAD = 163840
_ACC12 = 30080
_ACC3 = 10112
_WSC = 32
_NBLK = 25
_BLK = _N // _NBLK


def _sc_agg(n_chunks, chunks_per_core, tbl_rows, w, acc_rows, nbuf, batch):
    mesh = plsc.VectorSubcoreMesh(core_axis_name="c", subcore_axis_name="s")
    zrows = acc_rows // _TILES
    nrows = _EPAD // batch
    rpt = nrows // _TILES
    n_outer = rpt // _NB

    def body(tbl, gidx, sidx, zeros, out, gi_v, si_v, *rest):
        bufs = rest[:nbuf]
        acc = rest[nbuf]
        sems = rest[nbuf + 1:nbuf + 1 + nbuf]
        c = lax.axis_index("c")
        s = lax.axis_index("s")
        for cc in range(chunks_per_core):
            chunk = c * chunks_per_core + cc
            pltpu.sync_copy(zeros.at[pl.ds(s * zrows, zrows)],
                            acc.at[pl.ds(s * zrows, zrows)])
            plsc.subcore_barrier()

            def outer(i, carry):
                base = s * rpt + i * _NB
                pltpu.sync_copy(gidx.at[pl.ds(base, _NB)], gi_v)
                pltpu.sync_copy(sidx.at[pl.ds(base, _NB)], si_v)
                cps = {}
                for k in range(nbuf - 1):
                    cps[k] = pltpu.async_copy(
                        tbl.at[chunk].at[gi_v.at[k]], bufs[k], sems[k])
                for j in range(_NB):
                    cps[j].wait()
                    nxt = j + nbuf - 1
                    if nxt < _NB:
                        cps[nxt] = pltpu.async_copy(
                            tbl.at[chunk].at[gi_v.at[nxt]],
                            bufs[nxt % nbuf], sems[nxt % nbuf])
                    pltpu.sync_copy(bufs[j % nbuf], acc.at[si_v.at[j]],
                                    add=True)
                return carry

            lax.fori_loop(0, n_outer, outer, 0)
            plsc.subcore_barrier()
            pltpu.sync_copy(acc.at[pl.ds(s * zrows, zrows)],
                            out.at[chunk].at[pl.ds(s * zrows, zrows)])
            if chunks_per_core > 1:
                plsc.subcore_barrier()

    return pl.kernel(
        body, mesh=mesh,
        compiler_params=pltpu.CompilerParams(use_tc_tiling_on_sc=False),
        out_type=jax.ShapeDtypeStruct((n_chunks, acc_rows, w), jnp.float32),
        scratch_types=[
            pltpu.VMEM((_NB, batch), jnp.int32),
            pltpu.VMEM((_NB, batch), jnp.int32),
            *[pltpu.VMEM((batch, w), jnp.float32) for _ in range(nbuf)],
            pltpu.VMEM_SHARED((acc_rows, w), jnp.float32),
            *[pltpu.SemaphoreType.DMA for _ in range(nbuf)],
        ])


def _sc_agg3(n_chunks, chunks_per_core, tbl_rows, w, acc_rows, nbuf, batch):
    mesh = plsc.VectorSubcoreMesh(core_axis_name="c", subcore_axis_name="s")
    zrows = acc_rows // _TILES
    rpt = _EPAD // batch // _TILES

    def body(tbl, gidx, sidx, zeros, out, gi_v, si_v, *rest):
        bufs = rest[:nbuf]
        acc = rest[nbuf]
        sems = rest[nbuf + 1:nbuf + 1 + nbuf]
        c = lax.axis_index("c")
        s = lax.axis_index("s")
        for cc in range(chunks_per_core):
            chunk = c * chunks_per_core + cc
            pltpu.sync_copy(zeros.at[pl.ds(s * zrows, zrows)],
                            acc.at[pl.ds(s * zrows, zrows)])
            plsc.subcore_barrier()
            pltpu.sync_copy(gidx.at[s], gi_v)
            pltpu.sync_copy(sidx.at[s], si_v)
            cps = {}
            for k in range(nbuf - 1):
                cps[k] = pltpu.async_copy(
                    tbl.at[chunk].at[gi_v.at[k]], bufs[k], sems[k])
            for j in range(rpt):
                cps[j].wait()
                nxt = j + nbuf - 1
                if nxt < rpt:
                    cps[nxt] = pltpu.async_copy(
                        tbl.at[chunk].at[gi_v.at[nxt]],
                        bufs[nxt % nbuf], sems[nxt % nbuf])
                pltpu.sync_copy(bufs[j % nbuf], acc.at[si_v.at[j]], add=True)
            plsc.subcore_barrier()
            pltpu.sync_copy(acc.at[pl.ds(s * zrows, zrows)],
                            out.at[chunk].at[pl.ds(s * zrows, zrows)])
            if chunks_per_core > 1:
                plsc.subcore_barrier()

    return pl.kernel(
        body, mesh=mesh,
        compiler_params=pltpu.CompilerParams(use_tc_tiling_on_sc=False),
        out_type=jax.ShapeDtypeStruct((n_chunks, acc_rows, w), jnp.float32),
        scratch_types=[
            pltpu.VMEM((rpt, batch), jnp.int32),
            pltpu.VMEM((rpt, batch), jnp.int32),
            *[pltpu.VMEM((batch, w), jnp.float32) for _ in range(nbuf)],
            pltpu.VMEM_SHARED((acc_rows, w), jnp.float32),
            *[pltpu.SemaphoreType.DMA for _ in range(nbuf)],
        ])


def _full(shape):
    return pl.BlockSpec(shape, lambda *a: (0,) * len(shape))


def _tc_layer(n_chunks, din, dout, emit_sh):
    def body(a0_ref, a1_ref, a2_ref, x_ref, v_ref, comp_ref, wl_ref, b_ref,
             *outs):
        comp = comp_ref[...]
        v = v_ref[...]
        a = [jnp.concatenate([a_ref[cc] for cc in range(n_chunks)], axis=-1)
             for a_ref in (a0_ref, a1_ref, a2_ref)]
        acc = jnp.dot(x_ref[...], wl_ref[...], precision=lax.Precision.DEFAULT,
                      preferred_element_type=jnp.float32) + b_ref[...]
        for b in range(2):
            c_b = comp[0, b] * a[0] + comp[1, b] * a[1] + comp[2, b] * a[2]
            acc += jnp.dot(c_b, v[b], precision=lax.Precision.DEFAULT,
                           preferred_element_type=jnp.float32)
        h = jnp.maximum(acc, 0.0)
        outs[0][...] = h
        if emit_sh:
            for cc in range(dout // _WSC):
                outs[1][cc] = h[:, cc * _WSC:(cc + 1) * _WSC]

    if emit_sh:
        out_shape = [
            jax.ShapeDtypeStruct((_N, dout), jnp.float32),
            jax.ShapeDtypeStruct((dout // _WSC, _N, _WSC), jnp.float32),
        ]
        out_specs = [
            pl.BlockSpec((_BLK, dout), lambda i: (i, 0)),
            pl.BlockSpec((dout // _WSC, _BLK, _WSC), lambda i: (0, i, 0)),
        ]
    else:
        out_shape = jax.ShapeDtypeStruct((_N, dout), jnp.float32)
        out_specs = pl.BlockSpec((_BLK, dout), lambda i: (i, 0))

    return pl.pallas_call(
        body,
        grid=(_NBLK,),
        in_specs=[
            *[pl.BlockSpec((n_chunks, _BLK, din // n_chunks),
                           lambda i, r=r: (0, r * _NBLK + i, 0))
              for r in range(_R)],
            pl.BlockSpec((_BLK, din), lambda i: (i, 0)),
            _full((2, din, dout)),
            _full((_R, 2)),
            _full((din, dout)),
            _full((1, dout)),
        ],
        out_shape=out_shape,
        out_specs=out_specs,
    )


def _tc_rel3(din, dout):
    def body(x_ref, wcat_ref, comp_ref, b_ref, xw_ref, base_ref, y_ref):
        r = pl.program_id(1)
        comp = comp_ref[...]
        half = dout // 2

        @pl.when(r == 0)
        def _():
            y_ref[...] = jnp.dot(
                x_ref[...], wcat_ref[...], precision=lax.Precision.DEFAULT,
                preferred_element_type=jnp.float32)
            base_ref[...] = y_ref[:, :dout] + b_ref[...]

        c0 = jnp.where(r == 0, comp[0, 0], jnp.where(r == 1, comp[1, 0],
                                                     comp[2, 0]))
        c1 = jnp.where(r == 0, comp[0, 1], jnp.where(r == 1, comp[1, 1],
                                                     comp[2, 1]))
        m = c0 * y_ref[:, dout:2 * dout] + c1 * y_ref[:, 2 * dout:]
        xw_ref[0] = m[:, :half]
        xw_ref[1] = m[:, half:]

    return pl.pallas_call(
        body,
        grid=(_NBLK, _R),
        in_specs=[
            pl.BlockSpec((_BLK, din), lambda i, r: (i, 0)),
            _full((din, 3 * dout)),
            _full((_R, 2)),
            _full((1, dout)),
        ],
        out_shape=[
            jax.ShapeDtypeStruct((2, _R * _N, dout // 2), jnp.float32),
            jax.ShapeDtypeStruct((_N, dout), jnp.float32),
        ],
        out_specs=[
            pl.BlockSpec((2, _BLK, dout // 2),
                         lambda i, r: (0, r * _NBLK + i, 0)),
            pl.BlockSpec((_BLK, dout), lambda i, r: (i, 0)),
        ],
        scratch_shapes=[pltpu.VMEM((_BLK, 3 * dout), jnp.float32)],
    )


def _tc_final(dmid, dcls):
    def body(agg_ref, base_ref, wc_ref, bc_ref, out_ref, acc_ref):
        i = pl.program_id(0)
        h3 = jnp.maximum(
            jnp.concatenate([agg_ref[0], agg_ref[1]], axis=-1)
            + base_ref[...], 0.0)
        part = jnp.sum(h3, axis=0, keepdims=True)

        @pl.when(i == 0)
        def _():
            acc_ref[...] = part

        @pl.when(i > 0)
        def _():
            acc_ref[...] += part

        @pl.when(i == _NBLK - 1)
        def _():
            pooled = acc_ref[...] * (1.0 / _N)
            out_ref[...] = jnp.dot(pooled, wc_ref[...],
                                   preferred_element_type=jnp.float32) \
                + bc_ref[...]

    return pl.pallas_call(
        body,
        grid=(_NBLK,),
        in_specs=[
            pl.BlockSpec((2, _BLK, dmid // 2), lambda i: (0, i, 0)),
            pl.BlockSpec((_BLK, dmid), lambda i: (i, 0)),
            _full((dmid, dcls)),
            _full((1, dcls)),
        ],
        out_shape=jax.ShapeDtypeStruct((1, dcls), jnp.float32),
        out_specs=_full((1, dcls)),
        scratch_shapes=[pltpu.VMEM((1, dmid), jnp.float32)],
    )


def kernel(f, edge_index, e, V1, comp1, Wl1, b1, V2, comp2, Wl2, b2,
           V3, comp3, Wl3, b3, Wc, bc):
    src = edge_index[0]
    dst = edge_index[1]
    npad = _EPAD - _E
    pad = jnp.arange(npad, dtype=jnp.int32) % 16

    g12 = jnp.concatenate([src, pad]).reshape(_TILES, -1, 512)
    s12 = jnp.concatenate([e * _N + dst,
                           _R * _N + pad]).reshape(_TILES, -1, 512)
    g3 = jnp.concatenate([e * _N + src, pad]).reshape(_EPAD // 128, 128)
    s3 = jnp.concatenate([dst, _N + pad]).reshape(_EPAD // 128, 128)

    zeros12 = jnp.zeros((_ACC12, _WSC), jnp.float32)
    zeros3 = jnp.zeros((_ACC3, 128), jnp.float32)

    nc1 = 128 // _WSC
    f_sh = f.reshape(_N, nc1, _WSC).transpose(1, 0, 2)
    a1 = _sc_agg3(nc1, max(nc1 // 2, 1), _N, _WSC, _ACC12, 3, 512)(
        f_sh, g12, s12, zeros12)
    h1, h1_sh = _tc_layer(nc1, 128, 256, True)(
        a1, a1, a1, f, V1, comp1, Wl1, b1.reshape(1, -1))

    nc2 = 256 // _WSC
    a2 = _sc_agg3(nc2, nc2 // 2, _N, _WSC, _ACC12, 3, 512)(h1_sh, g12, s12,
                                                           zeros12)
    h2 = _tc_layer(nc2, 256, 512, False)(
        a2, a2, a2, h1, V2, comp2, Wl2, b2.reshape(1, -1))

    wcat3 = jnp.concatenate([Wl3, V3[0], V3[1]], axis=1)
    xw3, base3 = _tc_rel3(512, 256)(h2, wcat3, comp3, b3.reshape(1, -1))
    agg3 = _sc_agg(2, 1, _R * _N, 128, _ACC3, 2, 128)(xw3, g3, s3, zeros3)

    return _tc_final(256, 250)(agg3, base3, Wc, bc.reshape(1, -1))


# speedup vs baseline: 1.0047x; 1.0047x over previous
import functools

import jax
import jax.numpy as jnp
from jax import lax
from jax.experimental import pallas as pl
from jax.experimental.pallas import tpu as pltpu
from jax.experimental.pallas import tpu_sc as plsc

_N = 10000
_E = 160000
_R = 3
_NB = 8
_TILES = 16
_EPAD = 163840
_ACC12 = 30080
_ACC3 = 10112
_WSC = 32
_NBLK = 25
_BLK = _N // _NBLK


def _sc_agg(n_chunks, chunks_per_core, tbl_rows, w, acc_rows, nbuf, batch):
    mesh = plsc.VectorSubcoreMesh(core_axis_name="c", subcore_axis_name="s")
    zrows = acc_rows // _TILES
    nrows = _EPAD // batch
    rpt = nrows // _TILES
    n_outer = rpt // _NB

    def body(tbl, gidx, sidx, zeros, out, gi_v, si_v, *rest):
        bufs = rest[:nbuf]
        acc = rest[nbuf]
        sems = rest[nbuf + 1:nbuf + 1 + nbuf]
        c = lax.axis_index("c")
        s = lax.axis_index("s")
        for cc in range(chunks_per_core):
            chunk = c * chunks_per_core + cc
            pltpu.sync_copy(zeros.at[pl.ds(s * zrows, zrows)],
                            acc.at[pl.ds(s * zrows, zrows)])
            plsc.subcore_barrier()

            def outer(i, carry):
                base = s * rpt + i * _NB
                pltpu.sync_copy(gidx.at[pl.ds(base, _NB)], gi_v)
                pltpu.sync_copy(sidx.at[pl.ds(base, _NB)], si_v)
                cps = {}
                for k in range(nbuf - 1):
                    cps[k] = pltpu.async_copy(
                        tbl.at[chunk].at[gi_v.at[k]], bufs[k], sems[k])
                for j in range(_NB):
                    cps[j].wait()
                    nxt = j + nbuf - 1
                    if nxt < _NB:
                        cps[nxt] = pltpu.async_copy(
                            tbl.at[chunk].at[gi_v.at[nxt]],
                            bufs[nxt % nbuf], sems[nxt % nbuf])
                    pltpu.sync_copy(bufs[j % nbuf], acc.at[si_v.at[j]],
                                    add=True)
                return carry

            lax.fori_loop(0, n_outer, outer, 0)
            plsc.subcore_barrier()
            pltpu.sync_copy(acc.at[pl.ds(s * zrows, zrows)],
                            out.at[chunk].at[pl.ds(s * zrows, zrows)])
            if chunks_per_core > 1:
                plsc.subcore_barrier()

    return pl.kernel(
        body, mesh=mesh,
        compiler_params=pltpu.CompilerParams(use_tc_tiling_on_sc=False),
        out_type=jax.ShapeDtypeStruct((n_chunks, acc_rows, w), jnp.float32),
        scratch_types=[
            pltpu.VMEM((_NB, batch), jnp.int32),
            pltpu.VMEM((_NB, batch), jnp.int32),
            *[pltpu.VMEM((batch, w), jnp.float32) for _ in range(nbuf)],
            pltpu.VMEM_SHARED((acc_rows, w), jnp.float32),
            *[pltpu.SemaphoreType.DMA for _ in range(nbuf)],
        ])


def _sc_agg3(n_chunks, chunks_per_core, tbl_rows, w, acc_rows, nbuf, batch):
    mesh = plsc.VectorSubcoreMesh(core_axis_name="c", subcore_axis_name="s")
    zrows = acc_rows // _TILES
    rpt = _EPAD // batch // _TILES

    def body(tbl, gidx, sidx, zeros, out, gi_v, si_v, *rest):
        bufs = rest[:nbuf]
        acc = rest[nbuf]
        sems = rest[nbuf + 1:nbuf + 1 + nbuf]
        c = lax.axis_index("c")
        s = lax.axis_index("s")
        for cc in range(chunks_per_core):
            chunk = c * chunks_per_core + cc
            pltpu.sync_copy(zeros.at[pl.ds(s * zrows, zrows)],
                            acc.at[pl.ds(s * zrows, zrows)])
            plsc.subcore_barrier()
            pltpu.sync_copy(gidx.at[s], gi_v)
            pltpu.sync_copy(sidx.at[s], si_v)
            cps = {}
            for k in range(nbuf - 1):
                cps[k] = pltpu.async_copy(
                    tbl.at[chunk].at[gi_v.at[k]], bufs[k], sems[k])
            for j in range(rpt):
                cps[j].wait()
                nxt = j + nbuf - 1
                if nxt < rpt:
                    cps[nxt] = pltpu.async_copy(
                        tbl.at[chunk].at[gi_v.at[nxt]],
                        bufs[nxt % nbuf], sems[nxt % nbuf])
                pltpu.sync_copy(bufs[j % nbuf], acc.at[si_v.at[j]], add=True)
            plsc.subcore_barrier()
            pltpu.sync_copy(acc.at[pl.ds(s * zrows, zrows)],
                            out.at[chunk].at[pl.ds(s * zrows, zrows)])
            if chunks_per_core > 1:
                plsc.subcore_barrier()

    return pl.kernel(
        body, mesh=mesh,
        compiler_params=pltpu.CompilerParams(use_tc_tiling_on_sc=False),
        out_type=jax.ShapeDtypeStruct((n_chunks, acc_rows, w), jnp.float32),
        scratch_types=[
            pltpu.VMEM((rpt, batch), jnp.int32),
            pltpu.VMEM((rpt, batch), jnp.int32),
            *[pltpu.VMEM((batch, w), jnp.float32) for _ in range(nbuf)],
            pltpu.VMEM_SHARED((acc_rows, w), jnp.float32),
            *[pltpu.SemaphoreType.DMA for _ in range(nbuf)],
        ])


def _full(shape):
    return pl.BlockSpec(shape, lambda *a: (0,) * len(shape))


def _tc_selfloop(din, dout):
    def body(x_ref, wl_ref, b_ref, out_ref):
        out_ref[...] = jnp.dot(
            x_ref[...], wl_ref[...], precision=lax.Precision.DEFAULT,
            preferred_element_type=jnp.float32) + b_ref[...]

    return pl.pallas_call(
        body,
        grid=(_NBLK,),
        in_specs=[
            pl.BlockSpec((_BLK, din), lambda i: (i, 0)),
            _full((din, dout)),
            _full((1, dout)),
        ],
        out_shape=jax.ShapeDtypeStruct((_N, dout), jnp.float32),
        out_specs=pl.BlockSpec((_BLK, dout), lambda i: (i, 0)),
    )


def _tc_layer(n_chunks, din, dout, emit_sh):
    def body(a0_ref, a1_ref, a2_ref, base_ref, v_ref, comp_ref, *outs):
        comp = comp_ref[...]
        v = v_ref[...]
        acc = base_ref[...]
        for r, a_ref in enumerate((a0_ref, a1_ref, a2_ref)):
            w_r = comp[r, 0] * v[0] + comp[r, 1] * v[1]
            a_r = jnp.concatenate([a_ref[cc] for cc in range(n_chunks)],
                                  axis=-1)
            acc += jnp.dot(a_r, w_r, precision=lax.Precision.DEFAULT,
                           preferred_element_type=jnp.float32)
        h = jnp.maximum(acc, 0.0)
        outs[0][...] = h
        if emit_sh:
            for cc in range(dout // _WSC):
                outs[1][cc] = h[:, cc * _WSC:(cc + 1) * _WSC]

    if emit_sh:
        out_shape = [
            jax.ShapeDtypeStruct((_N, dout), jnp.float32),
            jax.ShapeDtypeStruct((dout // _WSC, _N, _WSC), jnp.float32),
        ]
        out_specs = [
            pl.BlockSpec((_BLK, dout), lambda i: (i, 0)),
            pl.BlockSpec((dout // _WSC, _BLK, _WSC), lambda i: (0, i, 0)),
        ]
    else:
        out_shape = jax.ShapeDtypeStruct((_N, dout), jnp.float32)
        out_specs = pl.BlockSpec((_BLK, dout), lambda i: (i, 0))

    return pl.pallas_call(
        body,
        grid=(_NBLK,),
        in_specs=[
            *[pl.BlockSpec((n_chunks, _BLK, din // n_chunks),
                           lambda i, r=r: (0, r * _NBLK + i, 0))
              for r in range(_R)],
            pl.BlockSpec((_BLK, dout), lambda i: (i, 0)),
            _full((2, din, dout)),
            _full((_R, 2)),
        ],
        out_shape=out_shape,
        out_specs=out_specs,
    )


def _tc_rel3(din, dout):
    def body(x_ref, v_ref, comp_ref, out_ref):
        r = pl.program_id(0)
        comp = comp_ref[...]
        c0 = jnp.where(r == 0, comp[0, 0], jnp.where(r == 1, comp[1, 0],
                                                     comp[2, 0]))
        c1 = jnp.where(r == 0, comp[0, 1], jnp.where(r == 1, comp[1, 1],
                                                     comp[2, 1]))
        v = v_ref[...]
        w_r = c0 * v[0] + c1 * v[1]
        m = jnp.dot(x_ref[...], w_r, precision=lax.Precision.DEFAULT,
                    preferred_element_type=jnp.float32)
        half = dout // 2
        out_ref[0] = m[:, :half]
        out_ref[1] = m[:, half:]

    return pl.pallas_call(
        body,
        grid=(_R, _NBLK),
        in_specs=[
            pl.BlockSpec((_BLK, din), lambda r, i: (i, 0)),
            _full((2, din, dout)),
            _full((_R, 2)),
        ],
        out_shape=jax.ShapeDtypeStruct((2, _R * _N, dout // 2), jnp.float32),
        out_specs=pl.BlockSpec((2, _BLK, dout // 2),
                               lambda r, i: (0, r * _NBLK + i, 0)),
    )


def _tc_final(dmid, dcls):
    def body(agg_ref, base_ref, wc_ref, bc_ref, out_ref, acc_ref):
        i = pl.program_id(0)
        h3 = jnp.maximum(
            jnp.concatenate([agg_ref[0], agg_ref[1]], axis=-1)
            + base_ref[...], 0.0)
        part = jnp.sum(h3, axis=0, keepdims=True)

        @pl.when(i == 0)
        def _():
            acc_ref[...] = part

        @pl.when(i > 0)
        def _():
            acc_ref[...] += part

        @pl.when(i == _NBLK - 1)
        def _():
            pooled = acc_ref[...] * (1.0 / _N)
            out_ref[...] = jnp.dot(pooled, wc_ref[...],
                                   preferred_element_type=jnp.float32) \
                + bc_ref[...]

    return pl.pallas_call(
        body,
        grid=(_NBLK,),
        in_specs=[
            pl.BlockSpec((2, _BLK, dmid // 2), lambda i: (0, i, 0)),
            pl.BlockSpec((_BLK, dmid), lambda i: (i, 0)),
            _full((dmid, dcls)),
            _full((1, dcls)),
        ],
        out_shape=jax.ShapeDtypeStruct((1, dcls), jnp.float32),
        out_specs=_full((1, dcls)),
        scratch_shapes=[pltpu.VMEM((1, dmid), jnp.float32)],
    )


def kernel(f, edge_index, e, V1, comp1, Wl1, b1, V2, comp2, Wl2, b2,
           V3, comp3, Wl3, b3, Wc, bc):
    src = edge_index[0]
    dst = edge_index[1]
    npad = _EPAD - _E
    pad = jnp.arange(npad, dtype=jnp.int32) % 16

    g12 = jnp.concatenate([src, pad]).reshape(_TILES, -1, 512)
    s12 = jnp.concatenate([e * _N + dst,
                           _R * _N + pad]).reshape(_TILES, -1, 512)
    g3 = jnp.concatenate([e * _N + src, pad]).reshape(_EPAD // 128, 128)
    s3 = jnp.concatenate([dst, _N + pad]).reshape(_EPAD // 128, 128)

    zeros12 = jnp.zeros((_ACC12, _WSC), jnp.float32)
    zeros3 = jnp.zeros((_ACC3, 128), jnp.float32)

    nc1 = 128 // _WSC
    f_sh = f.reshape(_N, nc1, _WSC).transpose(1, 0, 2)
    a1 = _sc_agg3(nc1, max(nc1 // 2, 1), _N, _WSC, _ACC12, 3, 512)(
        f_sh, g12, s12, zeros12)
    base1 = _tc_selfloop(128, 256)(f, Wl1, b1.reshape(1, -1))
    h1, h1_sh = _tc_layer(nc1, 128, 256, True)(a1, a1, a1, base1, V1, comp1)

    nc2 = 256 // _WSC
    a2 = _sc_agg3(nc2, nc2 // 2, _N, _WSC, _ACC12, 3, 512)(h1_sh, g12, s12,
                                                           zeros12)
    base2 = _tc_selfloop(256, 512)(h1, Wl2, b2.reshape(1, -1))
    h2 = _tc_layer(nc2, 256, 512, False)(a2, a2, a2, base2, V2, comp2)

    xw3 = _tc_rel3(512, 256)(h2, V3, comp3)
    agg3 = _sc_agg(2, 1, _R * _N, 128, _ACC3, 2, 128)(xw3, g3, s3, zeros3)
    base3 = _tc_selfloop(512, 256)(h2, Wl3, b3.reshape(1, -1))

    return _tc_final(256, 250)(agg3, base3, Wc, bc.reshape(1, -1))
